# baseline (speedup 1.0000x reference)
import functools

import jax
import jax.numpy as jnp
from jax import lax
from jax.experimental import pallas as pl
from jax.experimental.pallas import tpu as pltpu
from jax.experimental.pallas import tpu_sc as plsc

NUM_WORDS = 1000000
EMBED_DIMS = 64
NUM_COOCCUR_TYPES = 4
B_PAD = 8
BATCH = 16384

_info = plsc.get_sparse_core_info()
_NC = _info.num_cores
_NS = _info.num_subcores
_NW = _NC * _NS
_BPW = BATCH // _NW
_CHUNK = 128
_NCH = _BPW // _CHUNK

_mesh = plsc.VectorSubcoreMesh(core_axis_name="c", subcore_axis_name="s")


@functools.partial(
    pl.kernel,
    mesh=_mesh,
    out_type=(
        jax.ShapeDtypeStruct((BATCH, EMBED_DIMS), jnp.float32),
        jax.ShapeDtypeStruct((BATCH, B_PAD), jnp.float32),
    ),
    scratch_types=[
        pltpu.VMEM((_NCH, _CHUNK), jnp.int32),
        pltpu.VMEM((_BPW, EMBED_DIMS), jnp.float32),
        pltpu.VMEM((_BPW, B_PAD), jnp.float32),
        pltpu.SemaphoreType.DMA,
        pltpu.SemaphoreType.DMA,
    ],
    compiler_params=pltpu.CompilerParams(use_tc_tiling_on_sc=False),
)
def _embedding_gather(ids_hbm, w_hbm, b_hbm, w_out_hbm, b_out_hbm,
                      idx_v, wrows_v, brows_v, sem_w, sem_b):
    wid = lax.axis_index("s") * _NC + lax.axis_index("c")
    base = wid * _BPW
    pltpu.sync_copy(ids_hbm.at[pl.ds(wid * _NCH, _NCH)], idx_v)
    copies = []
    for j in range(_NCH):
        copies.append(pltpu.async_copy(
            w_hbm.at[idx_v.at[j]],
            wrows_v.at[pl.ds(j * _CHUNK, _CHUNK)], sem_w))
        copies.append(pltpu.async_copy(
            b_hbm.at[idx_v.at[j]],
            brows_v.at[pl.ds(j * _CHUNK, _CHUNK)], sem_b))
    for c in copies:
        c.wait()
    pltpu.sync_copy(wrows_v, w_out_hbm.at[pl.ds(base, _BPW)])
    pltpu.sync_copy(brows_v, b_out_hbm.at[pl.ds(base, _BPW)])


def kernel(ids, W, b):
    ids2 = ids.astype(jnp.int32).reshape(BATCH // _CHUNK, _CHUNK)
    b8 = jnp.pad(b, ((0, 0), (0, B_PAD - NUM_COOCCUR_TYPES)))
    w_out, b_out = _embedding_gather(ids2, W, b8)
    return (w_out, b_out[:, :NUM_COOCCUR_TYPES])


# speedup vs baseline: 3.7968x; 3.7968x over previous
import functools

import jax
import jax.numpy as jnp
from jax import lax
from jax.experimental import pallas as pl
from jax.experimental.pallas import tpu as pltpu
from jax.experimental.pallas import tpu_sc as plsc

NUM_WORDS = 1000000
EMBED_DIMS = 64
NUM_COOCCUR_TYPES = 4
LANES = 128
BATCH = 16384

_info = plsc.get_sparse_core_info()
_NC = _info.num_cores
_NS = _info.num_subcores
_NW = _NC * _NS
_BPW = BATCH // _NW
_CHUNK = 128
_NCH = _BPW // _CHUNK

_mesh = plsc.VectorSubcoreMesh(core_axis_name="c", subcore_axis_name="s")


@functools.partial(
    pl.kernel,
    mesh=_mesh,
    out_type=jax.ShapeDtypeStruct((BATCH, LANES), jnp.float32),
    scratch_types=[
        pltpu.VMEM((_NCH, _CHUNK), jnp.int32),
        pltpu.VMEM((_BPW, LANES), jnp.float32),
        pltpu.SemaphoreType.DMA,
    ],
)
def _w_gather(ids_hbm, w_hbm, out_hbm, idx_v, rows_v, sem):
    wid = lax.axis_index("s") * _NC + lax.axis_index("c")
    base = wid * _BPW
    pltpu.sync_copy(ids_hbm.at[pl.ds(wid * _NCH, _NCH)], idx_v)
    copies = []
    for j in range(_NCH):
        copies.append(pltpu.async_copy(
            w_hbm.at[idx_v.at[j]],
            rows_v.at[pl.ds(j * _CHUNK, _CHUNK)], sem))
    for c in copies:
        c.wait()
    pltpu.sync_copy(rows_v, out_hbm.at[pl.ds(base, _BPW)])


@functools.partial(
    pl.kernel,
    mesh=_mesh,
    out_type=jax.ShapeDtypeStruct((BATCH * NUM_COOCCUR_TYPES,), jnp.float32),
    scratch_types=[
        pltpu.VMEM((_NCH, _CHUNK), jnp.int32),
        pltpu.VMEM((NUM_COOCCUR_TYPES * _NCH, _CHUNK), jnp.int32),
        pltpu.VMEM((NUM_COOCCUR_TYPES * _BPW, 8), jnp.float32),
        pltpu.VMEM((_BPW * NUM_COOCCUR_TYPES,), jnp.float32),
        pltpu.SemaphoreType.DMA,
    ],
    compiler_params=pltpu.CompilerParams(use_tc_tiling_on_sc=False,
                                         needs_layout_passes=False),
)
def _b_gather(ids_hbm, bv_hbm, out_hbm, idx_v, idb_v, brows_v, pack_v, sem):
    wid = lax.axis_index("s") * _NC + lax.axis_index("c")
    pltpu.sync_copy(ids_hbm.at[pl.ds(wid * _NCH, _NCH)], idx_v)
    for c in range(_NCH):
        for g in range(_CHUNK // 16):
            v = idx_v[c, pl.ds(g * 16, 16)] >> 3
            for t in range(NUM_COOCCUR_TYPES):
                idb_v[t * _NCH + c, pl.ds(g * 16, 16)] = (
                    v + t * (NUM_WORDS // 8))
    copies = []
    for t in range(NUM_COOCCUR_TYPES):
        for c in range(_NCH):
            copies.append(pltpu.async_copy(
                bv_hbm.at[idb_v.at[t * _NCH + c]],
                brows_v.at[pl.ds((t * _NCH + c) * _CHUNK, _CHUNK)], sem))
    for c in copies:
        c.wait()
    iota = lax.iota(jnp.int32, 16)
    for c in range(_NCH):
        for g in range(_CHUNK // 16):
            ids16 = idx_v[c, pl.ds(g * 16, 16)]
            off = ids16 & 7
            dst_base = (c * _CHUNK + g * 16 + iota) * NUM_COOCCUR_TYPES
            for t in range(NUM_COOCCUR_TYPES):
                row = (t * _NCH + c) * _CHUNK + g * 16 + iota
                vals = plsc.load_gather(brows_v, [row, off])
                plsc.store_scatter(pack_v, [dst_base + t], vals)
    pltpu.sync_copy(
        pack_v,
        out_hbm.at[pl.ds(wid * _BPW * NUM_COOCCUR_TYPES,
                         _BPW * NUM_COOCCUR_TYPES)])


def kernel(ids, W, b):
    ids2 = ids.astype(jnp.int32).reshape(BATCH // _CHUNK, _CHUNK)
    w128 = jnp.pad(W, ((0, 0), (0, LANES - EMBED_DIMS)))
    bv = b.T.reshape(NUM_WORDS // 2, 2 * NUM_COOCCUR_TYPES)
    rows = _w_gather(ids2, w128)
    bflat = _b_gather(ids2, bv)
    return (rows[:, :EMBED_DIMS],
            bflat.reshape(BATCH, NUM_COOCCUR_TYPES))
